# baseline (speedup 1.0000x reference)
import functools

import jax
import jax.numpy as jnp
from jax import lax
from jax.experimental import pallas as pl
from jax.experimental.pallas import tpu as pltpu
from jax.experimental.pallas import tpu_sc as plsc

N = 10000
E = 160000
D_NODE = 128
D_EDGE = 16
H = 16
T = 3
S2S_STEPS = 12
B = 64

NW = 32
CH = 128
KC = 40
GCH = 512
GKC = 10
EP = NW * KC * CH
NPAD = 10016
RPS = NPAD // 16

_f32 = jnp.float32


def _prep_x_body(nf_ref, w_ref, b_ref, o_ref):
    x = jnp.dot(nf_ref[...], w_ref[...], preferred_element_type=_f32) + b_ref[...]
    rows = lax.broadcasted_iota(jnp.int32, (NPAD, 1), 0)
    o_ref[...] = jnp.where(rows < N, x, 0.0)


def _prep_eh_body(ef_ref, w1t_ref, b_ref, o_ref):
    o_ref[...] = jax.nn.relu(_dott(w1t_ref[...], ef_ref[...]) + b_ref[...])


MSG_BLK = 4096


def _msg_body(hs_ref, eh_ref, wpt_ref, btt_ref, o_ref):
    hs = hs_ref[...]
    ehT = eh_ref[...]
    uT = _dott(wpt_ref[...], hs)
    accT = _dott(btt_ref[...], hs)
    for k in range(H):
        accT = accT + ehT[k:k + 1, :] * uT[k * H:(k + 1) * H, :]
    cols = lax.broadcasted_iota(jnp.int32, (H, MSG_BLK), 1) + pl.program_id(0) * MSG_BLK
    msgT = jnp.where(cols < E, accT, 0.0)
    o_ref[...] = msgT.T


def _gru_body(m2_ref, x_ref, wihT_ref, whhT_ref, bihT_ref, bhhT_ref, o_ref):
    m = m2_ref[0] + m2_ref[1]
    x = x_ref[...]
    giT = _dott(wihT_ref[...], m) + bihT_ref[...]
    ghT = _dott(whhT_ref[...], x) + bhhT_ref[...]
    r = jax.nn.sigmoid(giT[:H, :] + ghT[:H, :])
    z = jax.nn.sigmoid(giT[H:2 * H, :] + ghT[H:2 * H, :])
    n = jnp.tanh(giT[2 * H:, :] + r * ghT[2 * H:, :])
    xnT = (1.0 - z) * n + z * x.T
    cols = lax.broadcasted_iota(jnp.int32, (H, NPAD), 1)
    o_ref[...] = jnp.where(cols < N, xnT, 0.0).T


def _dott(a, b):
    return lax.dot_general(a, b, (((1,), (1,)), ((), ())),
                           preferred_element_type=_f32)


def _s2s_body(x_ref, batch_ref, batchr_ref, wlih_ref, wlhh_ref, blih_ref,
              blhh_ref, wout_ref, bout_ref, o_ref):
    x = x_ref[...]
    bvec = batch_ref[...]
    cols = lax.broadcasted_iota(jnp.int32, (NPAD, B), 1)
    P = (bvec == cols).astype(_f32)
    rows = lax.broadcasted_iota(jnp.int32, (B, NPAD), 0)
    PT = (batchr_ref[...] == rows).astype(_f32)
    valid = (bvec < B).astype(_f32)

    q_star = jnp.zeros((B, 2 * H), _f32)
    h = jnp.zeros((B, H), _f32)
    c = jnp.zeros((B, H), _f32)
    for _ in range(S2S_STEPS):
        g = (jnp.dot(q_star, wlih_ref[...], preferred_element_type=_f32)
             + jnp.dot(h, wlhh_ref[...], preferred_element_type=_f32)
             + blih_ref[...] + blhh_ref[...])
        ig = jax.nn.sigmoid(g[:, :H])
        fg = jax.nn.sigmoid(g[:, H:2 * H])
        gg = jnp.tanh(g[:, 2 * H:3 * H])
        og = jax.nn.sigmoid(g[:, 3 * H:])
        c = fg * c + ig * gg
        h = og * jnp.tanh(c)
        q = h
        qg = jnp.dot(P, q, preferred_element_type=_f32)
        e = jnp.sum(x * qg, axis=1, keepdims=True)
        e_seg = jnp.where(P > 0.0, e, -1e30)
        e_max = jnp.max(e_seg, axis=0, keepdims=True)
        e_max_n = _dott(P, e_max)
        a = jnp.exp(e - e_max_n) * valid
        denom = jnp.dot(PT, a, preferred_element_type=_f32)
        denom_n = jnp.dot(P, denom, preferred_element_type=_f32)
        a = a / jnp.where(denom_n > 0.0, denom_n, 1.0)
        r = jnp.dot(PT, a * x, preferred_element_type=_f32)
        q_star = jnp.concatenate([q, r], axis=1)
    o_ref[...] = (jnp.dot(q_star[:, :H], wout_ref[...],
                          preferred_element_type=_f32) + bout_ref[...])


_MESH = plsc.VectorSubcoreMesh(core_axis_name="c", subcore_axis_name="s")


@functools.partial(
    pl.kernel,
    out_type=jax.ShapeDtypeStruct((NW, GKC, GCH, H), _f32),
    mesh=_MESH,
    compiler_params=pltpu.CompilerParams(use_tc_tiling_on_sc=False),
    scratch_types=[
        pltpu.VMEM((GKC, GCH), jnp.int32),
        pltpu.VMEM((GKC, GCH, H), _f32),
        pltpu.SemaphoreType.DMA,
    ],
)
def _sc_gather(x_hbm, idx_hbm, out_hbm, idx_v, rows_v, sem):
    c = lax.axis_index("c")
    s = lax.axis_index("s")
    wid = s * 2 + c
    pltpu.sync_copy(idx_hbm.at[wid], idx_v)

    def fire(j, carry):
        pltpu.make_async_copy(x_hbm.at[idx_v.at[j]], rows_v.at[j], sem).start()
        return carry

    lax.fori_loop(0, GKC, fire, 0)

    def drain(j, carry):
        pltpu.make_async_copy(x_hbm.at[idx_v.at[j]], rows_v.at[j], sem).wait()
        return carry

    lax.fori_loop(0, GKC, drain, 0)
    pltpu.sync_copy(rows_v, out_hbm.at[wid])


@functools.partial(
    pl.kernel,
    out_type=jax.ShapeDtypeStruct((2, NPAD, H), _f32),
    mesh=_MESH,
    compiler_params=pltpu.CompilerParams(use_tc_tiling_on_sc=False),
    scratch_types=[
        pltpu.VMEM_SHARED((NPAD, H), _f32),
        pltpu.VMEM((KC, CH), jnp.int32),
        pltpu.VMEM((KC, CH, H), _f32),
    ],
)
def _sc_scatter(msg_hbm, idx_hbm, zeros_hbm, out_hbm, shared, idx_v, msg_v):
    c = lax.axis_index("c")
    s = lax.axis_index("s")
    wid = s * 2 + c
    sl = pl.ds(s * RPS, RPS)
    pltpu.sync_copy(zeros_hbm.at[sl], shared.at[sl])
    pltpu.sync_copy(idx_hbm.at[wid], idx_v)
    pltpu.sync_copy(msg_hbm.at[wid], msg_v)
    plsc.subcore_barrier()

    def body(j, carry):
        pltpu.sync_copy(msg_v.at[j], shared.at[idx_v.at[j]], add=True)
        return carry

    lax.fori_loop(0, KC, body, 0)
    plsc.subcore_barrier()
    pltpu.sync_copy(shared.at[sl], out_hbm.at[c, sl])


def kernel(node_features, edge_features, Esrc, Etgt, batch,
           W_in, b_in, W_ee1, b_ee1, W_ee2, b_ee2,
           W_gih, W_ghh, b_gih, b_ghh,
           W_lih, W_lhh, b_lih, b_lhh,
           W_out, b_out):
    nf_pad = jnp.pad(node_features, ((0, NPAD - N), (0, 0)))
    ef_pad = jnp.pad(edge_features, ((0, EP - E), (0, 0)))
    esrc3 = jnp.pad(Esrc, (0, EP - E), constant_values=N).reshape(NW, GKC, GCH)
    etgt3 = jnp.pad(Etgt, (0, EP - E), constant_values=0).reshape(NW, KC, CH)
    batch2 = jnp.pad(batch, (0, NPAD - N), constant_values=B).reshape(NPAD, 1)
    batchr = jnp.pad(batch, (0, NPAD - N), constant_values=B).reshape(1, NPAD)
    zeros2 = jnp.zeros((NPAD, H), _f32)
    WpT = W_ee2.reshape(H * H, H)
    BtT = b_ee2.reshape(H, H)
    b_in2 = b_in.reshape(1, H)
    W_ee1T = W_ee1.T
    b_ee1c = b_ee1.reshape(H, 1)
    W_gihT = W_gih.T
    W_ghhT = W_ghh.T
    b_gihc = b_gih.reshape(3 * H, 1)
    b_ghhc = b_ghh.reshape(3 * H, 1)
    b_lih2 = b_lih.reshape(1, 4 * H)
    b_lhh2 = b_lhh.reshape(1, 4 * H)
    b_out2 = b_out.reshape(1, 1)

    x = pl.pallas_call(
        _prep_x_body,
        out_shape=jax.ShapeDtypeStruct((NPAD, H), _f32),
    )(nf_pad, W_in, b_in2)

    n_eh_blk = EP // 16384
    ehT = pl.pallas_call(
        _prep_eh_body,
        grid=(n_eh_blk,),
        in_specs=[
            pl.BlockSpec((16384, D_EDGE), lambda i: (i, 0)),
            pl.BlockSpec((H, D_EDGE), lambda i: (0, 0)),
            pl.BlockSpec((H, 1), lambda i: (0, 0)),
        ],
        out_specs=pl.BlockSpec((H, 16384), lambda i: (0, i)),
        out_shape=jax.ShapeDtypeStruct((H, EP), _f32),
    )(ef_pad, W_ee1T, b_ee1c)

    n_msg_blk = EP // MSG_BLK
    for _ in range(T):
        hsrc = _sc_gather(x, esrc3).reshape(EP, H)
        msg = pl.pallas_call(
            _msg_body,
            grid=(n_msg_blk,),
            in_specs=[
                pl.BlockSpec((MSG_BLK, H), lambda i: (i, 0)),
                pl.BlockSpec((H, MSG_BLK), lambda i: (0, i)),
                pl.BlockSpec((H * H, H), lambda i: (0, 0)),
                pl.BlockSpec((H, H), lambda i: (0, 0)),
            ],
            out_specs=pl.BlockSpec((MSG_BLK, H), lambda i: (i, 0)),
            out_shape=jax.ShapeDtypeStruct((EP, H), _f32),
        )(hsrc, ehT, WpT, BtT)
        m2 = _sc_scatter(msg.reshape(NW, KC, CH, H), etgt3, zeros2)
        x = pl.pallas_call(
            _gru_body,
            out_shape=jax.ShapeDtypeStruct((NPAD, H), _f32),
        )(m2, x, W_gihT, W_ghhT, b_gihc, b_ghhc)

    out = pl.pallas_call(
        _s2s_body,
        out_shape=jax.ShapeDtypeStruct((B, 1), _f32),
    )(x, batch2, batchr, W_lih, W_lhh, b_lih2, b_lhh2, W_out, b_out2)
    return out


# speedup vs baseline: 1.0003x; 1.0003x over previous
import functools

import jax
import jax.numpy as jnp
from jax import lax
from jax.experimental import pallas as pl
from jax.experimental.pallas import tpu as pltpu
from jax.experimental.pallas import tpu_sc as plsc

N = 10000
E = 160000
D_NODE = 128
D_EDGE = 16
H = 16
T = 3
S2S_STEPS = 12
B = 64

NW = 32
CH = 128
KC = 40
GCH = 128
GKC = 40
EP = NW * KC * CH
NPAD = 10016
RPS = NPAD // 16

_f32 = jnp.float32


def _prep_x_body(nf_ref, w_ref, b_ref, o_ref):
    x = jnp.dot(nf_ref[...], w_ref[...], preferred_element_type=_f32) + b_ref[...]
    rows = lax.broadcasted_iota(jnp.int32, (NPAD, 1), 0)
    o_ref[...] = jnp.where(rows < N, x, 0.0)


def _prep_eh_body(ef_ref, w1t_ref, b_ref, o_ref):
    o_ref[...] = jax.nn.relu(_dott(w1t_ref[...], ef_ref[...]) + b_ref[...])


MSG_BLK = 4096


def _msg_body(hs_ref, eh_ref, wpt_ref, btt_ref, o_ref):
    hs = hs_ref[...]
    ehT = eh_ref[...]
    uT = _dott(wpt_ref[...], hs)
    accT = _dott(btt_ref[...], hs)
    for k in range(H):
        accT = accT + ehT[k:k + 1, :] * uT[k * H:(k + 1) * H, :]
    cols = lax.broadcasted_iota(jnp.int32, (H, MSG_BLK), 1) + pl.program_id(0) * MSG_BLK
    msgT = jnp.where(cols < E, accT, 0.0)
    o_ref[...] = msgT.T


def _gru_body(m2_ref, x_ref, wihT_ref, whhT_ref, bihT_ref, bhhT_ref, o_ref):
    m = m2_ref[0] + m2_ref[1]
    x = x_ref[...]
    giT = _dott(wihT_ref[...], m) + bihT_ref[...]
    ghT = _dott(whhT_ref[...], x) + bhhT_ref[...]
    r = jax.nn.sigmoid(giT[:H, :] + ghT[:H, :])
    z = jax.nn.sigmoid(giT[H:2 * H, :] + ghT[H:2 * H, :])
    n = jnp.tanh(giT[2 * H:, :] + r * ghT[2 * H:, :])
    xnT = (1.0 - z) * n + z * x.T
    cols = lax.broadcasted_iota(jnp.int32, (H, NPAD), 1)
    o_ref[...] = jnp.where(cols < N, xnT, 0.0).T


def _dott(a, b):
    return lax.dot_general(a, b, (((1,), (1,)), ((), ())),
                           preferred_element_type=_f32)


def _s2s_body(x_ref, batch_ref, batchr_ref, wlih_ref, wlhh_ref, blih_ref,
              blhh_ref, wout_ref, bout_ref, o_ref):
    x = x_ref[...]
    bvec = batch_ref[...]
    cols = lax.broadcasted_iota(jnp.int32, (NPAD, B), 1)
    P = (bvec == cols).astype(_f32)
    rows = lax.broadcasted_iota(jnp.int32, (B, NPAD), 0)
    PT = (batchr_ref[...] == rows).astype(_f32)
    valid = (bvec < B).astype(_f32)

    q_star = jnp.zeros((B, 2 * H), _f32)
    h = jnp.zeros((B, H), _f32)
    c = jnp.zeros((B, H), _f32)
    for _ in range(S2S_STEPS):
        g = (jnp.dot(q_star, wlih_ref[...], preferred_element_type=_f32)
             + jnp.dot(h, wlhh_ref[...], preferred_element_type=_f32)
             + blih_ref[...] + blhh_ref[...])
        ig = jax.nn.sigmoid(g[:, :H])
        fg = jax.nn.sigmoid(g[:, H:2 * H])
        gg = jnp.tanh(g[:, 2 * H:3 * H])
        og = jax.nn.sigmoid(g[:, 3 * H:])
        c = fg * c + ig * gg
        h = og * jnp.tanh(c)
        q = h
        qg = jnp.dot(P, q, preferred_element_type=_f32)
        e = jnp.sum(x * qg, axis=1, keepdims=True)
        e_seg = jnp.where(P > 0.0, e, -1e30)
        e_max = jnp.max(e_seg, axis=0, keepdims=True)
        e_max_n = _dott(P, e_max)
        a = jnp.exp(e - e_max_n) * valid
        denom = jnp.dot(PT, a, preferred_element_type=_f32)
        denom_n = jnp.dot(P, denom, preferred_element_type=_f32)
        a = a / jnp.where(denom_n > 0.0, denom_n, 1.0)
        r = jnp.dot(PT, a * x, preferred_element_type=_f32)
        q_star = jnp.concatenate([q, r], axis=1)
    o_ref[...] = (jnp.dot(q_star[:, :H], wout_ref[...],
                          preferred_element_type=_f32) + bout_ref[...])


_MESH = plsc.VectorSubcoreMesh(core_axis_name="c", subcore_axis_name="s")


@functools.partial(
    pl.kernel,
    out_type=jax.ShapeDtypeStruct((NW, GKC, GCH, H), _f32),
    mesh=_MESH,
    compiler_params=pltpu.CompilerParams(use_tc_tiling_on_sc=False),
    scratch_types=[
        pltpu.VMEM((GKC, GCH), jnp.int32),
        pltpu.VMEM((GKC, GCH, H), _f32),
        pltpu.SemaphoreType.DMA,
    ],
)
def _sc_gather(x_hbm, idx_hbm, out_hbm, idx_v, rows_v, sem):
    c = lax.axis_index("c")
    s = lax.axis_index("s")
    wid = s * 2 + c
    pltpu.sync_copy(idx_hbm.at[wid], idx_v)

    def fire(j, carry):
        pltpu.make_async_copy(x_hbm.at[idx_v.at[j]], rows_v.at[j], sem).start()
        return carry

    lax.fori_loop(0, GKC, fire, 0)

    def drain(j, carry):
        pltpu.make_async_copy(x_hbm.at[idx_v.at[j]], rows_v.at[j], sem).wait()
        return carry

    lax.fori_loop(0, GKC, drain, 0)
    pltpu.sync_copy(rows_v, out_hbm.at[wid])


@functools.partial(
    pl.kernel,
    out_type=jax.ShapeDtypeStruct((2, NPAD, H), _f32),
    mesh=_MESH,
    compiler_params=pltpu.CompilerParams(use_tc_tiling_on_sc=False),
    scratch_types=[
        pltpu.VMEM_SHARED((NPAD, H), _f32),
        pltpu.VMEM((KC, CH), jnp.int32),
        pltpu.VMEM((KC, CH, H), _f32),
    ],
)
def _sc_scatter(msg_hbm, idx_hbm, zeros_hbm, out_hbm, shared, idx_v, msg_v):
    c = lax.axis_index("c")
    s = lax.axis_index("s")
    wid = s * 2 + c
    sl = pl.ds(s * RPS, RPS)
    pltpu.sync_copy(zeros_hbm.at[sl], shared.at[sl])
    pltpu.sync_copy(idx_hbm.at[wid], idx_v)
    pltpu.sync_copy(msg_hbm.at[wid], msg_v)
    plsc.subcore_barrier()

    def body(j, carry):
        pltpu.sync_copy(msg_v.at[j], shared.at[idx_v.at[j]], add=True)
        return carry

    lax.fori_loop(0, KC, body, 0)
    plsc.subcore_barrier()
    pltpu.sync_copy(shared.at[sl], out_hbm.at[c, sl])


def kernel(node_features, edge_features, Esrc, Etgt, batch,
           W_in, b_in, W_ee1, b_ee1, W_ee2, b_ee2,
           W_gih, W_ghh, b_gih, b_ghh,
           W_lih, W_lhh, b_lih, b_lhh,
           W_out, b_out):
    nf_pad = jnp.pad(node_features, ((0, NPAD - N), (0, 0)))
    ef_pad = jnp.pad(edge_features, ((0, EP - E), (0, 0)))
    esrc3 = jnp.pad(Esrc, (0, EP - E), constant_values=N).reshape(NW, GKC, GCH)
    etgt3 = jnp.pad(Etgt, (0, EP - E), constant_values=0).reshape(NW, KC, CH)
    batch2 = jnp.pad(batch, (0, NPAD - N), constant_values=B).reshape(NPAD, 1)
    batchr = jnp.pad(batch, (0, NPAD - N), constant_values=B).reshape(1, NPAD)
    zeros2 = jnp.zeros((NPAD, H), _f32)
    WpT = W_ee2.reshape(H * H, H)
    BtT = b_ee2.reshape(H, H)
    b_in2 = b_in.reshape(1, H)
    W_ee1T = W_ee1.T
    b_ee1c = b_ee1.reshape(H, 1)
    W_gihT = W_gih.T
    W_ghhT = W_ghh.T
    b_gihc = b_gih.reshape(3 * H, 1)
    b_ghhc = b_ghh.reshape(3 * H, 1)
    b_lih2 = b_lih.reshape(1, 4 * H)
    b_lhh2 = b_lhh.reshape(1, 4 * H)
    b_out2 = b_out.reshape(1, 1)

    x = pl.pallas_call(
        _prep_x_body,
        out_shape=jax.ShapeDtypeStruct((NPAD, H), _f32),
    )(nf_pad, W_in, b_in2)

    n_eh_blk = EP // 16384
    ehT = pl.pallas_call(
        _prep_eh_body,
        grid=(n_eh_blk,),
        in_specs=[
            pl.BlockSpec((16384, D_EDGE), lambda i: (i, 0)),
            pl.BlockSpec((H, D_EDGE), lambda i: (0, 0)),
            pl.BlockSpec((H, 1), lambda i: (0, 0)),
        ],
        out_specs=pl.BlockSpec((H, 16384), lambda i: (0, i)),
        out_shape=jax.ShapeDtypeStruct((H, EP), _f32),
    )(ef_pad, W_ee1T, b_ee1c)

    n_msg_blk = EP // MSG_BLK
    for _ in range(T):
        hsrc = _sc_gather(x, esrc3).reshape(EP, H)
        msg = pl.pallas_call(
            _msg_body,
            grid=(n_msg_blk,),
            in_specs=[
                pl.BlockSpec((MSG_BLK, H), lambda i: (i, 0)),
                pl.BlockSpec((H, MSG_BLK), lambda i: (0, i)),
                pl.BlockSpec((H * H, H), lambda i: (0, 0)),
                pl.BlockSpec((H, H), lambda i: (0, 0)),
            ],
            out_specs=pl.BlockSpec((MSG_BLK, H), lambda i: (i, 0)),
            out_shape=jax.ShapeDtypeStruct((EP, H), _f32),
        )(hsrc, ehT, WpT, BtT)
        m2 = _sc_scatter(msg.reshape(NW, KC, CH, H), etgt3, zeros2)
        x = pl.pallas_call(
            _gru_body,
            out_shape=jax.ShapeDtypeStruct((NPAD, H), _f32),
        )(m2, x, W_gihT, W_ghhT, b_gihc, b_ghhc)

    out = pl.pallas_call(
        _s2s_body,
        out_shape=jax.ShapeDtypeStruct((B, 1), _f32),
    )(x, batch2, batchr, W_lih, W_lhh, b_lih2, b_lhh2, W_out, b_out2)
    return out


# speedup vs baseline: 1.0056x; 1.0053x over previous
import functools

import jax
import jax.numpy as jnp
from jax import lax
from jax.experimental import pallas as pl
from jax.experimental.pallas import tpu as pltpu
from jax.experimental.pallas import tpu_sc as plsc

N = 10000
E = 160000
D_NODE = 128
D_EDGE = 16
H = 16
T = 3
S2S_STEPS = 12
B = 64

NW = 32
CH = 128
KC = 40
GCH = 128
GKC = 40
EP = NW * KC * CH
NPAD = 10016
RPS = NPAD // 16

_f32 = jnp.float32


def _prep_x_body(nf_ref, w_ref, b_ref, o_ref):
    x = jnp.dot(nf_ref[...], w_ref[...], preferred_element_type=_f32) + b_ref[...]
    rows = lax.broadcasted_iota(jnp.int32, (NPAD, 1), 0)
    o_ref[...] = jnp.where(rows < N, x, 0.0)


def _prep_eh_body(ef_ref, w1t_ref, b_ref, o_ref):
    o_ref[...] = jax.nn.relu(_dott(w1t_ref[...], ef_ref[...]) + b_ref[...])


MSG_BLK = 4096


def _msg_body(hs_ref, eh_ref, wpt_ref, btt_ref, o_ref):
    hs = hs_ref[...]
    ehT = eh_ref[...]
    uT = _dott(wpt_ref[...], hs)
    accT = _dott(btt_ref[...], hs)
    for k in range(H):
        accT = accT + ehT[k:k + 1, :] * uT[k * H:(k + 1) * H, :]
    cols = lax.broadcasted_iota(jnp.int32, (H, MSG_BLK), 1) + pl.program_id(0) * MSG_BLK
    msgT = jnp.where(cols < E, accT, 0.0)
    o_ref[...] = msgT.T


def _gru_body(m2_ref, x_ref, wihT_ref, whhT_ref, bihT_ref, bhhT_ref, o_ref):
    m = m2_ref[0] + m2_ref[1]
    x = x_ref[...]
    giT = _dott(wihT_ref[...], m) + bihT_ref[...]
    ghT = _dott(whhT_ref[...], x) + bhhT_ref[...]
    r = jax.nn.sigmoid(giT[:H, :] + ghT[:H, :])
    z = jax.nn.sigmoid(giT[H:2 * H, :] + ghT[H:2 * H, :])
    n = jnp.tanh(giT[2 * H:, :] + r * ghT[2 * H:, :])
    xnT = (1.0 - z) * n + z * x.T
    cols = lax.broadcasted_iota(jnp.int32, (H, NPAD), 1)
    o_ref[...] = jnp.where(cols < N, xnT, 0.0).T


def _dott(a, b):
    return lax.dot_general(a, b, (((1,), (1,)), ((), ())),
                           preferred_element_type=_f32)


def _s2s_body(m2_ref, xp_ref, wihT_ref, whhT_ref, bihT_ref, bhhT_ref,
              batch_ref, batchr_ref, wlih_ref, wlhh_ref, blih_ref,
              blhh_ref, wout_ref, bout_ref, o_ref):
    m = m2_ref[0] + m2_ref[1]
    xp = xp_ref[...]
    giT = _dott(wihT_ref[...], m) + bihT_ref[...]
    ghT = _dott(whhT_ref[...], xp) + bhhT_ref[...]
    rg = jax.nn.sigmoid(giT[:H, :] + ghT[:H, :])
    zg = jax.nn.sigmoid(giT[H:2 * H, :] + ghT[H:2 * H, :])
    ng = jnp.tanh(giT[2 * H:, :] + rg * ghT[2 * H:, :])
    xnT = (1.0 - zg) * ng + zg * xp.T
    colsn = lax.broadcasted_iota(jnp.int32, (H, NPAD), 1)
    x = jnp.where(colsn < N, xnT, 0.0).T
    bvec = batch_ref[...]
    cols = lax.broadcasted_iota(jnp.int32, (NPAD, B), 1)
    P = (bvec == cols).astype(_f32)
    rows = lax.broadcasted_iota(jnp.int32, (B, NPAD), 0)
    PT = (batchr_ref[...] == rows).astype(_f32)
    valid = (bvec < B).astype(_f32)

    q_star = jnp.zeros((B, 2 * H), _f32)
    h = jnp.zeros((B, H), _f32)
    c = jnp.zeros((B, H), _f32)
    for _ in range(S2S_STEPS):
        g = (jnp.dot(q_star, wlih_ref[...], preferred_element_type=_f32)
             + jnp.dot(h, wlhh_ref[...], preferred_element_type=_f32)
             + blih_ref[...] + blhh_ref[...])
        ig = jax.nn.sigmoid(g[:, :H])
        fg = jax.nn.sigmoid(g[:, H:2 * H])
        gg = jnp.tanh(g[:, 2 * H:3 * H])
        og = jax.nn.sigmoid(g[:, 3 * H:])
        c = fg * c + ig * gg
        h = og * jnp.tanh(c)
        q = h
        qg = jnp.dot(P, q, preferred_element_type=_f32)
        e = jnp.sum(x * qg, axis=1, keepdims=True)
        e_seg = jnp.where(P > 0.0, e, -1e30)
        e_max = jnp.max(e_seg, axis=0, keepdims=True)
        e_max_n = _dott(P, e_max)
        a = jnp.exp(e - e_max_n) * valid
        denom = jnp.dot(PT, a, preferred_element_type=_f32)
        denom_n = jnp.dot(P, denom, preferred_element_type=_f32)
        a = a / jnp.where(denom_n > 0.0, denom_n, 1.0)
        r = jnp.dot(PT, a * x, preferred_element_type=_f32)
        q_star = jnp.concatenate([q, r], axis=1)
    o_ref[...] = (jnp.dot(q_star[:, :H], wout_ref[...],
                          preferred_element_type=_f32) + bout_ref[...])


_MESH = plsc.VectorSubcoreMesh(core_axis_name="c", subcore_axis_name="s")


@functools.partial(
    pl.kernel,
    out_type=jax.ShapeDtypeStruct((NW, GKC, GCH, H), _f32),
    mesh=_MESH,
    compiler_params=pltpu.CompilerParams(use_tc_tiling_on_sc=False),
    scratch_types=[
        pltpu.VMEM((GKC, GCH), jnp.int32),
        pltpu.VMEM((GKC, GCH, H), _f32),
        pltpu.SemaphoreType.DMA,
    ],
)
def _sc_gather(x_hbm, idx_hbm, out_hbm, idx_v, rows_v, sem):
    c = lax.axis_index("c")
    s = lax.axis_index("s")
    wid = s * 2 + c
    pltpu.sync_copy(idx_hbm.at[wid], idx_v)

    def fire(j, carry):
        pltpu.make_async_copy(x_hbm.at[idx_v.at[j]], rows_v.at[j], sem).start()
        return carry

    lax.fori_loop(0, GKC, fire, 0)

    def drain(j, carry):
        pltpu.make_async_copy(x_hbm.at[idx_v.at[j]], rows_v.at[j], sem).wait()
        return carry

    lax.fori_loop(0, GKC, drain, 0)
    pltpu.sync_copy(rows_v, out_hbm.at[wid])


@functools.partial(
    pl.kernel,
    out_type=jax.ShapeDtypeStruct((2, NPAD, H), _f32),
    mesh=_MESH,
    compiler_params=pltpu.CompilerParams(use_tc_tiling_on_sc=False),
    scratch_types=[
        pltpu.VMEM_SHARED((NPAD, H), _f32),
        pltpu.VMEM((KC, CH), jnp.int32),
        pltpu.VMEM((KC, CH, H), _f32),
    ],
)
def _sc_scatter(msg_hbm, idx_hbm, zeros_hbm, out_hbm, shared, idx_v, msg_v):
    c = lax.axis_index("c")
    s = lax.axis_index("s")
    wid = s * 2 + c
    sl = pl.ds(s * RPS, RPS)
    pltpu.sync_copy(zeros_hbm.at[sl], shared.at[sl])
    pltpu.sync_copy(idx_hbm.at[wid], idx_v)
    pltpu.sync_copy(msg_hbm.at[wid], msg_v)
    plsc.subcore_barrier()

    def body(j, carry):
        pltpu.sync_copy(msg_v.at[j], shared.at[idx_v.at[j]], add=True)
        return carry

    lax.fori_loop(0, KC, body, 0)
    plsc.subcore_barrier()
    pltpu.sync_copy(shared.at[sl], out_hbm.at[c, sl])


def kernel(node_features, edge_features, Esrc, Etgt, batch,
           W_in, b_in, W_ee1, b_ee1, W_ee2, b_ee2,
           W_gih, W_ghh, b_gih, b_ghh,
           W_lih, W_lhh, b_lih, b_lhh,
           W_out, b_out):
    nf_pad = jnp.pad(node_features, ((0, NPAD - N), (0, 0)))
    ef_pad = jnp.pad(edge_features, ((0, EP - E), (0, 0)))
    esrc3 = jnp.pad(Esrc, (0, EP - E), constant_values=N).reshape(NW, GKC, GCH)
    etgt3 = jnp.pad(Etgt, (0, EP - E), constant_values=0).reshape(NW, KC, CH)
    batch2 = jnp.pad(batch, (0, NPAD - N), constant_values=B).reshape(NPAD, 1)
    batchr = jnp.pad(batch, (0, NPAD - N), constant_values=B).reshape(1, NPAD)
    zeros2 = jnp.zeros((NPAD, H), _f32)
    WpT = W_ee2.reshape(H * H, H)
    BtT = b_ee2.reshape(H, H)
    b_in2 = b_in.reshape(1, H)
    W_ee1T = W_ee1.T
    b_ee1c = b_ee1.reshape(H, 1)
    W_gihT = W_gih.T
    W_ghhT = W_ghh.T
    b_gihc = b_gih.reshape(3 * H, 1)
    b_ghhc = b_ghh.reshape(3 * H, 1)
    b_lih2 = b_lih.reshape(1, 4 * H)
    b_lhh2 = b_lhh.reshape(1, 4 * H)
    b_out2 = b_out.reshape(1, 1)

    x = pl.pallas_call(
        _prep_x_body,
        out_shape=jax.ShapeDtypeStruct((NPAD, H), _f32),
    )(nf_pad, W_in, b_in2)

    n_eh_blk = EP // 16384
    ehT = pl.pallas_call(
        _prep_eh_body,
        grid=(n_eh_blk,),
        in_specs=[
            pl.BlockSpec((16384, D_EDGE), lambda i: (i, 0)),
            pl.BlockSpec((H, D_EDGE), lambda i: (0, 0)),
            pl.BlockSpec((H, 1), lambda i: (0, 0)),
        ],
        out_specs=pl.BlockSpec((H, 16384), lambda i: (0, i)),
        out_shape=jax.ShapeDtypeStruct((H, EP), _f32),
    )(ef_pad, W_ee1T, b_ee1c)

    n_msg_blk = EP // MSG_BLK
    for t in range(T):
        hsrc = _sc_gather(x, esrc3).reshape(EP, H)
        msg = pl.pallas_call(
            _msg_body,
            grid=(n_msg_blk,),
            in_specs=[
                pl.BlockSpec((MSG_BLK, H), lambda i: (i, 0)),
                pl.BlockSpec((H, MSG_BLK), lambda i: (0, i)),
                pl.BlockSpec((H * H, H), lambda i: (0, 0)),
                pl.BlockSpec((H, H), lambda i: (0, 0)),
            ],
            out_specs=pl.BlockSpec((MSG_BLK, H), lambda i: (i, 0)),
            out_shape=jax.ShapeDtypeStruct((EP, H), _f32),
        )(hsrc, ehT, WpT, BtT)
        m2 = _sc_scatter(msg.reshape(NW, KC, CH, H), etgt3, zeros2)
        if t < T - 1:
            x = pl.pallas_call(
                _gru_body,
                out_shape=jax.ShapeDtypeStruct((NPAD, H), _f32),
            )(m2, x, W_gihT, W_ghhT, b_gihc, b_ghhc)

    out = pl.pallas_call(
        _s2s_body,
        out_shape=jax.ShapeDtypeStruct((B, 1), _f32),
    )(m2, x, W_gihT, W_ghhT, b_gihc, b_ghhc,
      batch2, batchr, W_lih, W_lhh, b_lih2, b_lhh2, W_out, b_out2)
    return out
